# baseline (speedup 1.0000x reference)
import jax
import jax.numpy as jnp
from jax import lax
from jax.experimental import pallas as pl
from jax.experimental.pallas import tpu as pltpu
from jax.experimental.pallas import tpu_sc as plsc

BATCH = 4
NPOS = 8192
HIDDEN = 768
NC, NS, L = 2, 16, 16
NW = NC * NS
POS_PER_W = NPOS // NW
CHUNK = 16
NCHUNK = POS_PER_W // CHUNK
NCOL = HIDDEN // L
NPHASE = NCHUNK * BATCH
RING = 4
SUPER = 8
NSUPER = NPHASE // SUPER
CBYTES = CHUNK * HIDDEN * 4


def _body(x_hbm, enc_hbm, out_hbm,
          xv0, xv1, xv2, xv3, ev0, ev1,
          isem0, isem1, isem2, isem3,
          osem0, osem1, osem2, osem3, esem0, esem1):
    xv = (xv0, xv1, xv2, xv3)
    ev = (ev0, ev1)
    isem = (isem0, isem1, isem2, isem3)
    osem = (osem0, osem1, osem2, osem3)
    esem = (esem0, esem1)

    wid = lax.axis_index("s") * NC + lax.axis_index("c")
    pos0 = wid * POS_PER_W

    def x_at(c, b):
        return x_hbm.at[b, pl.ds(pos0 + c * CHUNK, CHUNK)]

    def out_at(c, b):
        return out_hbm.at[b, pl.ds(pos0 + c * CHUNK, CHUNK)]

    def enc_at(c):
        return enc_hbm.at[pl.ds(pos0 + c * CHUNK, CHUNK)]

    pltpu.async_copy(enc_at(0), ev[0], esem[0])
    pltpu.async_copy(enc_at(1), ev[1], esem[1])
    pltpu.async_copy(x_at(0, 0), xv[0], isem[0])
    pltpu.async_copy(x_at(0, 1), xv[1], isem[1])

    def superphase(s, carry):
        for k in range(SUPER):
            p = k % RING
            cb = k // BATCH
            b = k % BATCH
            c = 2 * s + cb
            if k == 0:
                pltpu.make_async_copy(enc_at(c), ev[0], esem[0]).wait()
            if k == BATCH:
                pltpu.make_async_copy(enc_at(c), ev[1], esem[1]).wait()
            pltpu.make_async_copy(x_at(c, b), xv[p], isem[p]).wait()

            @plsc.parallel_loop(0, CHUNK * 4, 1, unroll=4)
            def _add(i):
                r = i >> 2
                j0 = (i & 3) * (NCOL // 4)
                es = [ev[cb][r, pl.ds((j0 + j) * L, L)]
                      for j in range(NCOL // 4)]
                for j in range(NCOL // 4):
                    plsc.addupdate(xv[p].at[r, pl.ds((j0 + j) * L, L)], es[j])

            pltpu.async_copy(xv[p], out_at(c, b), osem[p])

            q = (k + 2) % RING
            k2 = k + 2
            c2 = 2 * s + k2 // BATCH
            c2 = jnp.minimum(c2, NCHUNK - 1)
            b2 = k2 % BATCH
            if k < 2:
                @pl.when(s > 0)
                def _drain():
                    pltpu.make_async_copy(xv[q], out_at(c2, b2), osem[q]).wait()
            else:
                pltpu.make_async_copy(xv[q], out_at(c2, b2), osem[q]).wait()
            pltpu.async_copy(x_at(c2, b2), xv[q], isem[q])

            if k == SUPER - 1 - BATCH:
                pltpu.async_copy(
                    enc_at(jnp.minimum(2 * s + 2, NCHUNK - 1)),
                    ev[0], esem[0])
            if k == SUPER - 1:
                pltpu.async_copy(
                    enc_at(jnp.minimum(2 * s + 3, NCHUNK - 1)),
                    ev[1], esem[1])
        return carry

    lax.fori_loop(0, NSUPER, superphase, 0)

    pltpu.make_async_copy(enc_at(NCHUNK - 1), ev[0], esem[0]).wait()
    pltpu.make_async_copy(enc_at(NCHUNK - 1), ev[1], esem[1]).wait()
    pltpu.make_async_copy(x_at(NCHUNK - 1, 0), xv[0], isem[0]).wait()
    pltpu.make_async_copy(x_at(NCHUNK - 1, 1), xv[1], isem[1]).wait()
    pltpu.make_async_copy(xv[2], out_at(NCHUNK - 1, 2), osem[2]).wait()
    pltpu.make_async_copy(xv[3], out_at(NCHUNK - 1, 3), osem[3]).wait()


_sc_add = pl.kernel(
    _body,
    out_type=jax.ShapeDtypeStruct((BATCH, NPOS, HIDDEN), jnp.float32),
    mesh=plsc.VectorSubcoreMesh(
        core_axis_name="c", subcore_axis_name="s", num_cores=NC, num_subcores=NS
    ),
    scratch_types=(
        [pltpu.VMEM((CHUNK, HIDDEN), jnp.float32)] * (RING + 2)
        + [pltpu.SemaphoreType.DMA] * (RING + RING + 2)
    ),
)


@jax.jit
def kernel(x, enc_weight):
    return _sc_add(x, enc_weight)


# speedup vs baseline: 1.0179x; 1.0179x over previous
import jax
import jax.numpy as jnp
from jax import lax
from jax.experimental import pallas as pl
from jax.experimental.pallas import tpu as pltpu
from jax.experimental.pallas import tpu_sc as plsc

BATCH = 4
NPOS = 8192
HIDDEN = 768
NC, NS, L = 2, 16, 16
NW = NC * NS
POS_PER_W = NPOS // NW
CHUNK = 16
NCHUNK = POS_PER_W // CHUNK
NCOL = HIDDEN // L
NPHASE = NCHUNK * BATCH
RING = 6
LEAD = 3


def _body(x_hbm, enc_hbm, out_hbm, *refs):
    xv = refs[:RING]
    ev = refs[RING:RING + 2]
    isem = refs[RING + 2:2 * RING + 2]
    osem = refs[2 * RING + 2:3 * RING + 2]
    esem = refs[3 * RING + 2:3 * RING + 4]

    wid = lax.axis_index("s") * NC + lax.axis_index("c")
    pos0 = wid * POS_PER_W

    def x_slice(t):
        c, b = divmod(t, BATCH)
        return (b, pl.ds(pos0 + c * CHUNK, CHUNK))

    def enc_slice(c):
        return pl.ds(pos0 + c * CHUNK, CHUNK)

    enc_desc = [
        pltpu.async_copy(enc_hbm.at[enc_slice(0)], ev[0], esem[0]),
        pltpu.async_copy(enc_hbm.at[enc_slice(1)], ev[1], esem[1]),
    ]
    in_desc = [None] * NPHASE
    out_desc = [None] * NPHASE
    for t in range(LEAD):
        in_desc[t] = pltpu.async_copy(
            x_hbm.at[x_slice(t)], xv[t % RING], isem[t % RING])

    for t in range(NPHASE):
        c, b = divmod(t, BATCH)
        p = t % RING
        cb = c % 2
        if b == 0:
            enc_desc[cb].wait()
            if 1 <= c < NCHUNK - 1:
                enc_desc[1 - cb] = pltpu.async_copy(
                    enc_hbm.at[enc_slice(c + 1)], ev[1 - cb], esem[1 - cb])
        in_desc[t].wait()

        @plsc.parallel_loop(0, CHUNK * 4, 1, unroll=2)
        def _add(i):
            r = i >> 2
            j0 = (i & 3) * (NCOL // 4)
            es = [ev[cb][r, pl.ds((j0 + j) * L, L)] for j in range(NCOL // 4)]
            for j in range(NCOL // 4):
                plsc.addupdate(xv[p].at[r, pl.ds((j0 + j) * L, L)], es[j])

        out_desc[t] = pltpu.async_copy(
            xv[p], out_hbm.at[x_slice(t)], osem[p])
        if t + LEAD < NPHASE:
            tq = t + LEAD - RING
            if tq >= 0:
                out_desc[tq].wait()
                out_desc[tq] = None
            q = (t + LEAD) % RING
            in_desc[t + LEAD] = pltpu.async_copy(
                x_hbm.at[x_slice(t + LEAD)], xv[q], isem[q])

    for t in range(NPHASE):
        if out_desc[t] is not None:
            out_desc[t].wait()


_sc_add = pl.kernel(
    _body,
    out_type=jax.ShapeDtypeStruct((BATCH, NPOS, HIDDEN), jnp.float32),
    mesh=plsc.VectorSubcoreMesh(
        core_axis_name="c", subcore_axis_name="s", num_cores=NC, num_subcores=NS
    ),
    scratch_types=(
        [pltpu.VMEM((CHUNK, HIDDEN), jnp.float32)] * (RING + 2)
        + [pltpu.SemaphoreType.DMA] * (2 * RING + 2)
    ),
)


@jax.jit
def kernel(x, enc_weight):
    return _sc_add(x, enc_weight)
